# baseline (speedup 1.0000x reference)
import functools
import jax
import jax.numpy as jnp
from jax import lax
from jax.experimental import pallas as pl
from jax.experimental.pallas import tpu as pltpu
from jax.experimental.pallas import tpu_sc as plsc

N_NODES = 100000
NUM_GRAPHS = 512
D_FEAT = 128
TRASH = NUM_GRAPHS

NC = 2
NS = 16
NW = NC * NS
CHUNK = 128
NCHUNK = 25
NBUF = 7
AHEAD = 2
ZROWS = NUM_GRAPHS // NS


def _start(w, j):
    return 3125 * w + 125 * j - lax.rem(5 * w + 5 * j, 8)


def _sc_body(nodes_hbm, ids_hbm, ones_hbm, zrow_hbm, zcnt_hbm,
             part_hbm, cnt_hbm,
             idx_v, ones_v, zc_v, zbuf, bufs, acc_sh, cnt_sh,
             ld_sems, st_sems, cnt_sem, idx_sem):
    c = lax.axis_index("c")
    s = lax.axis_index("s")
    wid = s * NC + c

    starts = [pl.multiple_of(_start(wid, j), 8) for j in range(NCHUNK)]

    idx_h = [
        pltpu.async_copy(ids_hbm.at[pl.ds(starts[j], CHUNK)], idx_v.at[j],
                         idx_sem)
        for j in range(NCHUNK)
    ]
    ones_h = pltpu.async_copy(ones_hbm, ones_v, idx_sem)
    ld_h = [None] * NCHUNK
    st_h = [None] * NCHUNK
    for j in range(min(AHEAD, NCHUNK)):
        ld_h[j] = pltpu.async_copy(
            nodes_hbm.at[pl.ds(starts[j], CHUNK)], bufs.at[j % NBUF],
            ld_sems.at[j % NBUF])

    pltpu.sync_copy(zrow_hbm, zbuf)
    pltpu.sync_copy(zbuf, acc_sh.at[pl.ds(s * ZROWS, ZROWS)])

    @pl.when(s == 0)
    def _init():
        pltpu.sync_copy(zcnt_hbm, zc_v)
        pltpu.sync_copy(zc_v, cnt_sh.at[pl.ds(0, NUM_GRAPHS)])

    plsc.subcore_barrier()

    ones_h.wait()
    for h in idx_h:
        h.wait()

    lane = lax.iota(jnp.int32, 16)
    for j in range(NCHUNK):
        if j > 0:
            prev = _start(wid, j - 1)
        else:
            prev = jnp.where(wid == 0, starts[0] - CHUNK,
                             _start(wid - 1, NCHUNK - 1))
        ndup = prev + CHUNK - starts[j]
        v = idx_v[j, pl.ds(0, 16)]
        idx_v[j, pl.ds(0, 16)] = jnp.where(lane < ndup, TRASH, v)

    cnt_h = [None] * NCHUNK
    for j in range(NCHUNK):
        b = j % NBUF
        ld_h[j].wait()
        st_h[j] = pltpu.async_copy(bufs.at[b], acc_sh.at[idx_v.at[j]],
                                   st_sems.at[b], add=True)
        cnt_h[j] = pltpu.async_copy(ones_v.at[0], cnt_sh.at[idx_v.at[j]],
                                    cnt_sem, add=True)
        nxt = j + AHEAD
        if nxt < NCHUNK:
            if nxt >= NBUF:
                st_h[nxt - NBUF].wait()
            ld_h[nxt] = pltpu.async_copy(
                nodes_hbm.at[pl.ds(starts[nxt], CHUNK)], bufs.at[nxt % NBUF],
                ld_sems.at[nxt % NBUF])
    for j in range(max(NCHUNK - NBUF, 0), NCHUNK):
        st_h[j].wait()
    for h in cnt_h:
        h.wait()

    plsc.subcore_barrier()

    pltpu.sync_copy(acc_sh.at[pl.ds(s * ZROWS, ZROWS)],
                    part_hbm.at[c, pl.ds(s * ZROWS, ZROWS)])

    @pl.when(s == 0)
    def _flush():
        pltpu.sync_copy(cnt_sh.at[pl.ds(0, NUM_GRAPHS)], cnt_hbm.at[c])


def _segment_partials(node_states, ids, ones2, zrow, zcnt):
    mesh = plsc.VectorSubcoreMesh(core_axis_name="c", subcore_axis_name="s")
    f = functools.partial(
        pl.kernel,
        mesh=mesh,
        out_type=[
            jax.ShapeDtypeStruct((NC, NUM_GRAPHS, D_FEAT), jnp.float32),
            jax.ShapeDtypeStruct((NC, NUM_GRAPHS), jnp.float32),
        ],
        scratch_types=[
            pltpu.VMEM((NCHUNK, CHUNK), jnp.int32),
            pltpu.VMEM((1, CHUNK), jnp.float32),
            pltpu.VMEM((NUM_GRAPHS,), jnp.float32),
            pltpu.VMEM((ZROWS, D_FEAT), jnp.float32),
            pltpu.VMEM((NBUF, CHUNK, D_FEAT), jnp.float32),
            pltpu.VMEM_SHARED((NUM_GRAPHS + 8, D_FEAT), jnp.float32),
            pltpu.VMEM_SHARED((NUM_GRAPHS + 8,), jnp.float32),
            pltpu.SemaphoreType.DMA((NBUF,)),
            pltpu.SemaphoreType.DMA((NBUF,)),
            pltpu.SemaphoreType.DMA,
            pltpu.SemaphoreType.DMA,
        ],
    )(_sc_body)
    return f(node_states, ids, ones2, zrow, zcnt)


def _tc_finish_body(part_ref, cnt_ref, ctx_ref, w_ref, b_ref, out_ref):
    summed = part_ref[0] + part_ref[1]
    total = cnt_ref[0] + cnt_ref[1]
    r = (1.0 / jnp.maximum(total, 1.0))[:, None]
    pooled = summed * r
    w_top = w_ref[0:D_FEAT, :]
    w_bot = w_ref[D_FEAT:2 * D_FEAT, :]
    z = lax.dot_general(ctx_ref[...], w_top, (((1,), (0,)), ((), ())),
                        preferred_element_type=jnp.float32)
    z += lax.dot_general(pooled, w_bot, (((1,), (0,)), ((), ())),
                         preferred_element_type=jnp.float32)
    out_ref[...] = jnp.maximum(z + b_ref[...], 0.0)


def _tc_finish(part, cnt, context_state, W, b2):
    return pl.pallas_call(
        _tc_finish_body,
        out_shape=jax.ShapeDtypeStruct((NUM_GRAPHS, D_FEAT), jnp.float32),
    )(part, cnt, context_state, W, b2)


def kernel(node_states, context_state, node_graph_ids, W, b):
    ids = node_graph_ids.astype(jnp.int32)
    ones2 = jnp.ones((1, CHUNK), jnp.float32)
    zrow = jnp.zeros((ZROWS, D_FEAT), jnp.float32)
    zcnt = jnp.zeros((NUM_GRAPHS,), jnp.float32)

    part, cnt = _segment_partials(node_states, ids, ones2, zrow, zcnt)
    b2 = b.reshape(1, D_FEAT)
    return _tc_finish(part, cnt, context_state, W, b2)


# speedup vs baseline: 1.0271x; 1.0271x over previous
import functools
import jax
import jax.numpy as jnp
from jax import lax
from jax.experimental import pallas as pl
from jax.experimental.pallas import tpu as pltpu
from jax.experimental.pallas import tpu_sc as plsc

N_NODES = 100000
NUM_GRAPHS = 512
D_FEAT = 128
TRASH = NUM_GRAPHS

NC = 2
NS = 16
NW = NC * NS
CHUNK = 128
NCHUNK = 25
NBUF = 7
AHEAD = 3
ZROWS = NUM_GRAPHS // NS


def _start(w, j):
    return 3125 * w + 125 * j - lax.rem(5 * w + 5 * j, 8)


def _sc_body(nodes_hbm, ids_hbm, ones_hbm, zrow_hbm, zcnt_hbm,
             part_hbm, cnt_hbm,
             idx_v, ones_v, zc_v, zbuf, bufs, acc_sh, cnt_sh,
             ld_sems, st_sems, cnt_sem, idx_sem):
    c = lax.axis_index("c")
    s = lax.axis_index("s")
    wid = s * NC + c

    starts = [pl.multiple_of(_start(wid, j), 8) for j in range(NCHUNK)]

    idx_h = [
        pltpu.async_copy(ids_hbm.at[pl.ds(starts[j], CHUNK)], idx_v.at[j],
                         idx_sem)
        for j in range(NCHUNK)
    ]
    ones_h = pltpu.async_copy(ones_hbm, ones_v, idx_sem)
    ld_h = [None] * NCHUNK
    st_h = [None] * NCHUNK
    for j in range(min(AHEAD, NCHUNK)):
        ld_h[j] = pltpu.async_copy(
            nodes_hbm.at[pl.ds(starts[j], CHUNK)], bufs.at[j % NBUF],
            ld_sems.at[j % NBUF])

    pltpu.sync_copy(zrow_hbm, zbuf)
    pltpu.sync_copy(zbuf, acc_sh.at[pl.ds(s * ZROWS, ZROWS)])

    @pl.when(s == 0)
    def _init():
        pltpu.sync_copy(zcnt_hbm, zc_v)
        pltpu.sync_copy(zc_v, cnt_sh.at[pl.ds(0, NUM_GRAPHS)])

    plsc.subcore_barrier()

    ones_h.wait()
    for h in idx_h:
        h.wait()

    lane = lax.iota(jnp.int32, 16)
    for j in range(NCHUNK):
        if j > 0:
            prev = _start(wid, j - 1)
        else:
            prev = jnp.where(wid == 0, starts[0] - CHUNK,
                             _start(wid - 1, NCHUNK - 1))
        ndup = prev + CHUNK - starts[j]
        v = idx_v[j, pl.ds(0, 16)]
        idx_v[j, pl.ds(0, 16)] = jnp.where(lane < ndup, TRASH, v)

    cnt_h = [None] * NCHUNK
    for j in range(NCHUNK):
        b = j % NBUF
        ld_h[j].wait()
        st_h[j] = pltpu.async_copy(bufs.at[b], acc_sh.at[idx_v.at[j]],
                                   st_sems.at[b], add=True)
        cnt_h[j] = pltpu.async_copy(ones_v.at[0], cnt_sh.at[idx_v.at[j]],
                                    cnt_sem, add=True)
        nxt = j + AHEAD
        if nxt < NCHUNK:
            if nxt >= NBUF:
                st_h[nxt - NBUF].wait()
            ld_h[nxt] = pltpu.async_copy(
                nodes_hbm.at[pl.ds(starts[nxt], CHUNK)], bufs.at[nxt % NBUF],
                ld_sems.at[nxt % NBUF])
    for j in range(max(NCHUNK - NBUF, 0), NCHUNK):
        st_h[j].wait()
    for h in cnt_h:
        h.wait()

    plsc.subcore_barrier()

    pltpu.sync_copy(acc_sh.at[pl.ds(s * ZROWS, ZROWS)],
                    part_hbm.at[c, pl.ds(s * ZROWS, ZROWS)])

    @pl.when(s == 0)
    def _flush():
        pltpu.sync_copy(cnt_sh.at[pl.ds(0, NUM_GRAPHS)], cnt_hbm.at[c])


def _segment_partials(node_states, ids, ones2, zrow, zcnt):
    mesh = plsc.VectorSubcoreMesh(core_axis_name="c", subcore_axis_name="s")
    f = functools.partial(
        pl.kernel,
        mesh=mesh,
        out_type=[
            jax.ShapeDtypeStruct((NC, NUM_GRAPHS, D_FEAT), jnp.float32),
            jax.ShapeDtypeStruct((NC, NUM_GRAPHS), jnp.float32),
        ],
        scratch_types=[
            pltpu.VMEM((NCHUNK, CHUNK), jnp.int32),
            pltpu.VMEM((1, CHUNK), jnp.float32),
            pltpu.VMEM((NUM_GRAPHS,), jnp.float32),
            pltpu.VMEM((ZROWS, D_FEAT), jnp.float32),
            pltpu.VMEM((NBUF, CHUNK, D_FEAT), jnp.float32),
            pltpu.VMEM_SHARED((NUM_GRAPHS + 8, D_FEAT), jnp.float32),
            pltpu.VMEM_SHARED((NUM_GRAPHS + 8,), jnp.float32),
            pltpu.SemaphoreType.DMA((NBUF,)),
            pltpu.SemaphoreType.DMA((NBUF,)),
            pltpu.SemaphoreType.DMA,
            pltpu.SemaphoreType.DMA,
        ],
    )(_sc_body)
    return f(node_states, ids, ones2, zrow, zcnt)


def _tc_finish_body(part_ref, cnt_ref, ctx_ref, w_ref, b_ref, out_ref):
    summed = part_ref[0] + part_ref[1]
    total = cnt_ref[0] + cnt_ref[1]
    r = (1.0 / jnp.maximum(total, 1.0))[:, None]
    pooled = summed * r
    w_top = w_ref[0:D_FEAT, :]
    w_bot = w_ref[D_FEAT:2 * D_FEAT, :]
    z = lax.dot_general(ctx_ref[...], w_top, (((1,), (0,)), ((), ())),
                        preferred_element_type=jnp.float32)
    z += lax.dot_general(pooled, w_bot, (((1,), (0,)), ((), ())),
                         preferred_element_type=jnp.float32)
    out_ref[...] = jnp.maximum(z + b_ref[...], 0.0)


def _tc_finish(part, cnt, context_state, W, b2):
    return pl.pallas_call(
        _tc_finish_body,
        out_shape=jax.ShapeDtypeStruct((NUM_GRAPHS, D_FEAT), jnp.float32),
    )(part, cnt, context_state, W, b2)


def kernel(node_states, context_state, node_graph_ids, W, b):
    ids = node_graph_ids.astype(jnp.int32)
    ones2 = jnp.ones((1, CHUNK), jnp.float32)
    zrow = jnp.zeros((ZROWS, D_FEAT), jnp.float32)
    zcnt = jnp.zeros((NUM_GRAPHS,), jnp.float32)

    part, cnt = _segment_partials(node_states, ids, ones2, zrow, zcnt)
    b2 = b.reshape(1, D_FEAT)
    return _tc_finish(part, cnt, context_state, W, b2)


# speedup vs baseline: 1.1050x; 1.0758x over previous
import functools
import jax
import jax.numpy as jnp
from jax import lax
from jax.experimental import pallas as pl
from jax.experimental.pallas import tpu as pltpu
from jax.experimental.pallas import tpu_sc as plsc

N_NODES = 100000
NUM_GRAPHS = 512
D_FEAT = 128
TRASH = NUM_GRAPHS

NC = 2
NS = 16
NW = NC * NS
CHUNK = 128
NCHUNK = 25
NBUF = 7
AHEAD = 3
ZROWS = NUM_GRAPHS // NS


def _start(w, j):
    return 3125 * w + 125 * j - lax.rem(5 * w + 5 * j, 8)


def _sc_body(nodes_hbm, ids_hbm, ones_hbm, zrow_hbm, zcnt_hbm,
             part_hbm, cnt_hbm,
             idx_v, ones_v, zc_v, zbuf, bufs, acc_sh, cnt_sh,
             ld_sems, st_sems, cnt_sem, idx_sem):
    c = lax.axis_index("c")
    s = lax.axis_index("s")
    wid = s * NC + c

    starts = [pl.multiple_of(_start(wid, j), 8) for j in range(NCHUNK)]

    idx_h = [
        pltpu.async_copy(ids_hbm.at[pl.ds(starts[j], CHUNK)], idx_v.at[j],
                         idx_sem)
        for j in range(NCHUNK)
    ]
    ones_h = pltpu.async_copy(ones_hbm, ones_v, idx_sem)
    ld_h = [None] * NCHUNK
    st_h = [None] * NCHUNK
    for j in range(min(AHEAD, NCHUNK)):
        ld_h[j] = pltpu.async_copy(
            nodes_hbm.at[pl.ds(starts[j], CHUNK)], bufs.at[j % NBUF],
            ld_sems.at[j % NBUF])

    pltpu.sync_copy(zrow_hbm, zbuf)
    pltpu.sync_copy(zbuf, acc_sh.at[pl.ds(s * ZROWS, ZROWS)])

    @pl.when(s == 0)
    def _init():
        pltpu.sync_copy(zcnt_hbm, zc_v)
        pltpu.sync_copy(zc_v, cnt_sh.at[pl.ds(0, NUM_GRAPHS)])

    plsc.subcore_barrier()

    ones_h.wait()
    for h in idx_h:
        h.wait()

    lane = lax.iota(jnp.int32, 16)
    for j in range(NCHUNK):
        if j > 0:
            prev = _start(wid, j - 1)
        else:
            prev = jnp.where(wid == 0, starts[0] - CHUNK,
                             _start(wid - 1, NCHUNK - 1))
        ndup = prev + CHUNK - starts[j]
        v = idx_v[j, pl.ds(0, 16)]
        idx_v[j, pl.ds(0, 16)] = jnp.where(lane < ndup, TRASH, v)

    cnt_h = [
        pltpu.async_copy(ones_v.at[0], cnt_sh.at[idx_v.at[j]], cnt_sem,
                         add=True)
        for j in range(NCHUNK)
    ]

    for j in range(NCHUNK):
        b = j % NBUF
        ld_h[j].wait()
        st_h[j] = pltpu.async_copy(bufs.at[b], acc_sh.at[idx_v.at[j]],
                                   st_sems.at[b], add=True)
        nxt = j + AHEAD
        if nxt < NCHUNK:
            if nxt >= NBUF:
                st_h[nxt - NBUF].wait()
            ld_h[nxt] = pltpu.async_copy(
                nodes_hbm.at[pl.ds(starts[nxt], CHUNK)], bufs.at[nxt % NBUF],
                ld_sems.at[nxt % NBUF])
    for j in range(max(NCHUNK - NBUF, 0), NCHUNK):
        st_h[j].wait()
    for h in cnt_h:
        h.wait()

    plsc.subcore_barrier()

    pltpu.sync_copy(acc_sh.at[pl.ds(s * ZROWS, ZROWS)],
                    part_hbm.at[c, pl.ds(s * ZROWS, ZROWS)])

    @pl.when(s == 0)
    def _flush():
        pltpu.sync_copy(cnt_sh.at[pl.ds(0, NUM_GRAPHS)], cnt_hbm.at[c])


def _segment_partials(node_states, ids, ones2, zrow, zcnt):
    mesh = plsc.VectorSubcoreMesh(core_axis_name="c", subcore_axis_name="s")
    f = functools.partial(
        pl.kernel,
        mesh=mesh,
        out_type=[
            jax.ShapeDtypeStruct((NC, NUM_GRAPHS, D_FEAT), jnp.float32),
            jax.ShapeDtypeStruct((NC, NUM_GRAPHS), jnp.float32),
        ],
        scratch_types=[
            pltpu.VMEM((NCHUNK, CHUNK), jnp.int32),
            pltpu.VMEM((1, CHUNK), jnp.float32),
            pltpu.VMEM((NUM_GRAPHS,), jnp.float32),
            pltpu.VMEM((ZROWS, D_FEAT), jnp.float32),
            pltpu.VMEM((NBUF, CHUNK, D_FEAT), jnp.float32),
            pltpu.VMEM_SHARED((NUM_GRAPHS + 8, D_FEAT), jnp.float32),
            pltpu.VMEM_SHARED((NUM_GRAPHS + 8,), jnp.float32),
            pltpu.SemaphoreType.DMA((NBUF,)),
            pltpu.SemaphoreType.DMA((NBUF,)),
            pltpu.SemaphoreType.DMA,
            pltpu.SemaphoreType.DMA,
        ],
    )(_sc_body)
    return f(node_states, ids, ones2, zrow, zcnt)


def _tc_finish_body(part_ref, cnt_ref, ctx_ref, w_ref, b_ref, out_ref):
    summed = part_ref[0] + part_ref[1]
    total = cnt_ref[0] + cnt_ref[1]
    r = (1.0 / jnp.maximum(total, 1.0))[:, None]
    pooled = summed * r
    w_top = w_ref[0:D_FEAT, :]
    w_bot = w_ref[D_FEAT:2 * D_FEAT, :]
    z = lax.dot_general(ctx_ref[...], w_top, (((1,), (0,)), ((), ())),
                        preferred_element_type=jnp.float32)
    z += lax.dot_general(pooled, w_bot, (((1,), (0,)), ((), ())),
                         preferred_element_type=jnp.float32)
    out_ref[...] = jnp.maximum(z + b_ref[...], 0.0)


def _tc_finish(part, cnt, context_state, W, b2):
    return pl.pallas_call(
        _tc_finish_body,
        out_shape=jax.ShapeDtypeStruct((NUM_GRAPHS, D_FEAT), jnp.float32),
    )(part, cnt, context_state, W, b2)


def kernel(node_states, context_state, node_graph_ids, W, b):
    ids = node_graph_ids.astype(jnp.int32)
    ones2 = jnp.ones((1, CHUNK), jnp.float32)
    zrow = jnp.zeros((ZROWS, D_FEAT), jnp.float32)
    zcnt = jnp.zeros((NUM_GRAPHS,), jnp.float32)

    part, cnt = _segment_partials(node_states, ids, ones2, zrow, zcnt)
    b2 = b.reshape(1, D_FEAT)
    return _tc_finish(part, cnt, context_state, W, b2)
